# baseline (speedup 1.0000x reference)
import functools

import jax
import jax.numpy as jnp
from jax import lax
from jax.experimental import pallas as pl
from jax.experimental.pallas import tpu as pltpu
from jax.experimental.pallas import tpu_sc as plsc

_N, _D = 4096, 128
_BQ = 512
_IDXM = 4095
_NW = 32
_NPART = 2
_QP = _N // _NPART
_RPW = _QP // _NW


_BIAS = 1 << 23


def _keys_body(q_ref, k_ref, o_ref, w_ref, sqk_ref, *, qoff=0):
    qi = pl.program_id(0) + qoff
    q = q_ref[...]
    k = k_ref[...]

    @pl.when(qi == 0)
    def _():
        sqk_ref[...] = jnp.sum(k * k, axis=1, keepdims=True)

    g = lax.dot_general(
        k, q * 2.0, (((1,), (1,)), ((), ())),
        preferred_element_type=jnp.float32)
    sq_q = jnp.sum(q * q, axis=1, keepdims=True).T
    d2 = (sqk_ref[...] - g) + sq_q

    rows = lax.broadcasted_iota(jnp.int32, (k.shape[0], 1), 0)
    cols = lax.broadcasted_iota(jnp.int32, (1, _BQ), 1) + qi * _BQ
    d2 = jnp.where(rows == cols, 0.0, d2)

    bits = lax.bitcast_convert_type(d2, jnp.int32)
    keys = lax.bitcast_convert_type(
        (bits & jnp.int32(~_IDXM)) + (rows + _BIAS), jnp.float32)
    inf = jnp.float32(jnp.inf)
    m1 = jnp.min(keys, axis=0, keepdims=True)
    k2 = jnp.where(keys == m1, inf, keys)
    m2 = jnp.min(k2, axis=0, keepdims=True)
    k3 = jnp.where(k2 == m2, inf, k2)
    m3 = jnp.min(k3, axis=0, keepdims=True)

    def unpack(m):
        mb = lax.bitcast_convert_type(m, jnp.int32) - _BIAS
        d2m = lax.bitcast_convert_type(mb & jnp.int32(~_IDXM), jnp.float32)
        return mb & jnp.int32(_IDXM), 1.0 / jnp.maximum(d2m, 1e-16)

    for j, m in enumerate((m1, m2, m3)):
        idx, wts = unpack(m)
        o_ref[0, j:j + 1, :] = idx
        w_ref[0, j:j + 1, :] = wts


def _topk_keys(x, part):
    n, d = x.shape
    nblk = _QP // _BQ
    return pl.pallas_call(
        functools.partial(_keys_body, qoff=part * nblk),
        grid=(nblk,),
        in_specs=[
            pl.BlockSpec((_BQ, d), lambda i: (i + part * nblk, 0)),
            pl.BlockSpec((n, d), lambda i: (0, 0)),
        ],
        out_specs=[
            pl.BlockSpec((1, 3, _BQ), lambda i: (i, 0, 0)),
            pl.BlockSpec((1, 3, _BQ), lambda i: (i, 0, 0)),
        ],
        out_shape=[
            jax.ShapeDtypeStruct((nblk, 3, _BQ), jnp.int32),
            jax.ShapeDtypeStruct((nblk, 3, _BQ), jnp.float32),
        ],
        scratch_shapes=[pltpu.VMEM((n, 1), jnp.float32)],
    )(x, x)


def _sc_body(idx_hbm, wts_hbm, x_hbm, out_hbm, idx_v, w_v, g_v, out_v, sem):
    wid = lax.axis_index("s") * 2 + lax.axis_index("c")
    base = wid * _RPW
    kbase = (base // _BQ) * 3 * _BQ + base % _BQ

    for j in range(3):
        pltpu.sync_copy(idx_hbm.at[pl.ds(kbase + j * _BQ, _RPW)],
                        idx_v.at[j])
        pltpu.sync_copy(wts_hbm.at[pl.ds(kbase + j * _BQ, _RPW)],
                        w_v.at[j])

    copies = [
        pltpu.async_copy(x_hbm.at[idx_v.at[j]], g_v.at[j], sem)
        for j in range(3)
    ]
    for c in copies:
        c.wait()

    for s in range(_RPW // 16):
        sl = pl.ds(s * 16, 16)
        w0, w1, w2 = w_v[0, sl], w_v[1, sl], w_v[2, sl]
        inv = 1.0 / (w0 + w1 + w2)
        w_v[0, sl] = w0 * inv
        w_v[1, sl] = w1 * inv
        w_v[2, sl] = w2 * inv

    def group_body(g, carry):
        gb = g * 16
        wa = [w_v[j, pl.ds(gb, 16)] for j in range(3)]
        for i in range(16):
            r = gb + i
            a0, a1, a2 = wa[0][i], wa[1][i], wa[2][i]
            for s in range(_D // 16):
                sl = pl.ds(s * 16, 16)
                out_v[r, sl] = (g_v[0, r, sl] * a0 + g_v[1, r, sl] * a1
                                + g_v[2, r, sl] * a2)
        return carry

    lax.fori_loop(0, _RPW // 16, group_body, 0)

    pltpu.sync_copy(out_v, out_hbm.at[pl.ds(base, _RPW)])


@functools.cache
def _sc_interpolate():
    return functools.partial(
        pl.kernel,
        mesh=plsc.VectorSubcoreMesh(core_axis_name="c", subcore_axis_name="s"),
        out_type=jax.ShapeDtypeStruct((_QP, _D), jnp.float32),
        scratch_types=[
            pltpu.VMEM((3, _RPW), jnp.int32),
            pltpu.VMEM((3, _RPW), jnp.float32),
            pltpu.VMEM((3, _RPW, _D), jnp.float32),
            pltpu.VMEM((_RPW, _D), jnp.float32),
            pltpu.SemaphoreType.DMA,
        ],
    )(_sc_body)


@jax.jit
def kernel(x):
    outs = []
    for p in range(_NPART):
        idx, wts = _topk_keys(x, p)
        outs.append(_sc_interpolate()(idx.reshape(-1), wts.reshape(-1), x))
    return jnp.concatenate(outs, axis=0)


# speedup vs baseline: 1.1183x; 1.1183x over previous
import functools

import jax
import jax.numpy as jnp
from jax import lax
from jax.experimental import pallas as pl
from jax.experimental.pallas import tpu as pltpu
from jax.experimental.pallas import tpu_sc as plsc

_N, _D = 4096, 128
_BQ = 512
_IDXM = 4095
_NW = 32
_NPART = 1
_QP = _N // _NPART
_RPW = _QP // _NW


_BIAS = 1 << 23


def _keys_body(q_ref, k_ref, o_ref, sqk_ref, *, qoff=0):
    qi = pl.program_id(0) + qoff
    q = q_ref[...]
    k = k_ref[...]

    @pl.when(qi == 0)
    def _():
        sqk_ref[...] = jnp.sum(k * k, axis=1, keepdims=True)

    g = lax.dot_general(
        k, q * 2.0, (((1,), (1,)), ((), ())),
        preferred_element_type=jnp.float32)
    sq_q = jnp.sum(q * q, axis=1, keepdims=True).T
    d2 = (sqk_ref[...] - g) + sq_q

    rows = lax.broadcasted_iota(jnp.int32, (k.shape[0], 1), 0)
    cols = lax.broadcasted_iota(jnp.int32, (1, _BQ), 1) + qi * _BQ
    d2 = jnp.where(rows == cols, 0.0, d2)

    bits = lax.bitcast_convert_type(d2, jnp.int32)
    keys = lax.bitcast_convert_type(
        (bits & jnp.int32(~_IDXM)) + (rows + _BIAS), jnp.float32)
    inf = jnp.float32(jnp.inf)
    m1 = jnp.min(keys, axis=0, keepdims=True)
    k2 = jnp.where(keys == m1, inf, keys)
    m2 = jnp.min(k2, axis=0, keepdims=True)
    k3 = jnp.where(k2 == m2, inf, k2)
    m3 = jnp.min(k3, axis=0, keepdims=True)

    def unpack(m):
        mb = lax.bitcast_convert_type(m, jnp.int32) - _BIAS
        d2m = lax.bitcast_convert_type(mb & jnp.int32(~_IDXM), jnp.float32)
        idx = (mb & jnp.int32(_IDXM)).astype(jnp.float32)
        return idx, 1.0 / jnp.maximum(d2m, 1e-16)

    iw = [unpack(m) for m in (m1, m2, m3)]
    inv = 1.0 / (iw[0][1] + iw[1][1] + iw[2][1])
    for j in range(3):
        o_ref[0, j:j + 1, :] = iw[j][0]
        o_ref[0, 3 + j:4 + j, :] = iw[j][1] * inv


def _topk_keys(x, part):
    n, d = x.shape
    nblk = _QP // _BQ
    return pl.pallas_call(
        functools.partial(_keys_body, qoff=part * nblk),
        grid=(nblk,),
        in_specs=[
            pl.BlockSpec((_BQ, d), lambda i: (i + part * nblk, 0)),
            pl.BlockSpec((n, d), lambda i: (0, 0)),
        ],
        out_specs=pl.BlockSpec((1, 6, _BQ), lambda i: (i, 0, 0)),
        out_shape=jax.ShapeDtypeStruct((nblk, 6, _BQ), jnp.float32),
        scratch_shapes=[pltpu.VMEM((n, 1), jnp.float32)],
    )(x, x)


def _sc_body(iw_hbm, x_hbm, out_hbm, stage_v, idx_v, g_v, out_v, sem):
    wid = lax.axis_index("s") * 2 + lax.axis_index("c")
    base = wid * _RPW
    blk = base // _BQ
    off = base % _BQ

    pltpu.sync_copy(
        iw_hbm.at[pl.ds(blk, 1), pl.ds(0, 6), pl.ds(off, _RPW)], stage_v)

    for j in range(3):
        for s in range(_RPW // 16):
            sl = pl.ds(s * 16, 16)
            idx_v[j, sl] = stage_v[0, j, sl].astype(jnp.int32)

    copies = [
        pltpu.async_copy(x_hbm.at[idx_v.at[j]], g_v.at[j], sem)
        for j in range(3)
    ]
    for c in copies:
        c.wait()

    def group_body(g, carry):
        gb = g * 16
        wa = [stage_v[0, 3 + j, pl.ds(gb, 16)] for j in range(3)]
        for i in range(16):
            r = gb + i
            a0, a1, a2 = wa[0][i], wa[1][i], wa[2][i]
            for s in range(_D // 16):
                sl = pl.ds(s * 16, 16)
                out_v[r, sl] = (g_v[0, r, sl] * a0 + g_v[1, r, sl] * a1
                                + g_v[2, r, sl] * a2)
        return carry

    lax.fori_loop(0, _RPW // 16, group_body, 0)

    pltpu.sync_copy(out_v, out_hbm.at[pl.ds(base, _RPW)])


@functools.cache
def _sc_interpolate():
    return functools.partial(
        pl.kernel,
        mesh=plsc.VectorSubcoreMesh(core_axis_name="c", subcore_axis_name="s"),
        out_type=jax.ShapeDtypeStruct((_QP, _D), jnp.float32),
        scratch_types=[
            pltpu.VMEM((1, 6, _RPW), jnp.float32),
            pltpu.VMEM((3, _RPW), jnp.int32),
            pltpu.VMEM((3, _RPW, _D), jnp.float32),
            pltpu.VMEM((_RPW, _D), jnp.float32),
            pltpu.SemaphoreType.DMA,
        ],
    )(_sc_body)


@jax.jit
def kernel(x):
    outs = []
    for p in range(_NPART):
        iw = _topk_keys(x, p)
        outs.append(_sc_interpolate()(iw, x))
    return outs[0] if _NPART == 1 else jnp.concatenate(outs, axis=0)


# speedup vs baseline: 1.2675x; 1.1334x over previous
import functools

import jax
import jax.numpy as jnp
from jax import lax
from jax.experimental import pallas as pl
from jax.experimental.pallas import tpu as pltpu
from jax.experimental.pallas import tpu_sc as plsc

_N, _D = 4096, 128
_BQ = 512
_IDXM = 4095
_NW = 32
_NPART = 1
_QP = _N // _NPART
_RPW = _QP // _NW


_BIAS = 1 << 23


def _keys_body(q_ref, k_ref, o_ref, sqk_ref, *, qoff=0):
    qi = pl.program_id(0) + qoff
    q = q_ref[...]
    k = k_ref[...]

    @pl.when(qi == 0)
    def _():
        sqk_ref[...] = jnp.sum(k * k, axis=1, keepdims=True)

    g = lax.dot_general(
        k, q * 2.0, (((1,), (1,)), ((), ())),
        preferred_element_type=jnp.float32)
    sq_q = jnp.sum(q * q, axis=1, keepdims=True).T
    d2 = (sqk_ref[...] - g) + sq_q

    rows = lax.broadcasted_iota(jnp.int32, (k.shape[0], 1), 0)
    cols = lax.broadcasted_iota(jnp.int32, (1, _BQ), 1) + qi * _BQ

    bits = lax.bitcast_convert_type(d2, jnp.int32)
    keys = lax.bitcast_convert_type(
        (bits & jnp.int32(~_IDXM)) + (rows + _BIAS), jnp.float32)
    inf = jnp.float32(jnp.inf)
    k2 = jnp.where(rows == cols, inf, keys)
    m2 = jnp.min(k2, axis=0, keepdims=True)
    k3 = jnp.where(k2 == m2, inf, k2)
    m3 = jnp.min(k3, axis=0, keepdims=True)

    def unpack(m):
        mb = lax.bitcast_convert_type(m, jnp.int32) - _BIAS
        d2m = lax.bitcast_convert_type(mb & jnp.int32(~_IDXM), jnp.float32)
        idx = (mb & jnp.int32(_IDXM)).astype(jnp.float32)
        return idx, 1.0 / jnp.maximum(d2m, 1e-16)

    i2, w2 = unpack(m2)
    i3, w3 = unpack(m3)
    w1 = jnp.full(i2.shape, 1e16, jnp.float32)
    inv = 1.0 / (w1 + w2 + w3)
    o_ref[0, 0:1, :] = cols.astype(jnp.float32)
    o_ref[0, 1:2, :] = i2
    o_ref[0, 2:3, :] = i3
    o_ref[0, 3:4, :] = w1 * inv
    o_ref[0, 4:5, :] = w2 * inv
    o_ref[0, 5:6, :] = w3 * inv


def _topk_keys(x, part):
    n, d = x.shape
    nblk = _QP // _BQ
    return pl.pallas_call(
        functools.partial(_keys_body, qoff=part * nblk),
        grid=(nblk,),
        in_specs=[
            pl.BlockSpec((_BQ, d), lambda i: (i + part * nblk, 0)),
            pl.BlockSpec((n, d), lambda i: (0, 0)),
        ],
        out_specs=pl.BlockSpec((1, 6, _BQ), lambda i: (i, 0, 0)),
        out_shape=jax.ShapeDtypeStruct((nblk, 6, _BQ), jnp.float32),
        scratch_shapes=[pltpu.VMEM((n, 1), jnp.float32)],
    )(x, x)


def _sc_body(iw_hbm, x_hbm, out_hbm, stage_v, idx_v, g_v, out_v, sem):
    wid = lax.axis_index("s") * 2 + lax.axis_index("c")
    base = wid * _RPW
    blk = base // _BQ
    off = base % _BQ

    pltpu.sync_copy(
        iw_hbm.at[pl.ds(blk, 1), pl.ds(0, 6), pl.ds(off, _RPW)], stage_v)

    for j in range(3):
        for s in range(_RPW // 16):
            sl = pl.ds(s * 16, 16)
            idx_v[j, sl] = stage_v[0, j, sl].astype(jnp.int32)

    copies = [
        pltpu.async_copy(x_hbm.at[idx_v.at[j]], g_v.at[j], sem)
        for j in range(3)
    ]
    for c in copies:
        c.wait()

    def group_body(g, carry):
        gb = g * 16
        wa = [stage_v[0, 3 + j, pl.ds(gb, 16)] for j in range(3)]
        for i in range(16):
            r = gb + i
            a0, a1, a2 = wa[0][i], wa[1][i], wa[2][i]
            for s in range(_D // 16):
                sl = pl.ds(s * 16, 16)
                out_v[r, sl] = (g_v[0, r, sl] * a0 + g_v[1, r, sl] * a1
                                + g_v[2, r, sl] * a2)
        return carry

    lax.fori_loop(0, _RPW // 16, group_body, 0)

    pltpu.sync_copy(out_v, out_hbm.at[pl.ds(base, _RPW)])


@functools.cache
def _sc_interpolate():
    return functools.partial(
        pl.kernel,
        mesh=plsc.VectorSubcoreMesh(core_axis_name="c", subcore_axis_name="s"),
        out_type=jax.ShapeDtypeStruct((_QP, _D), jnp.float32),
        scratch_types=[
            pltpu.VMEM((1, 6, _RPW), jnp.float32),
            pltpu.VMEM((3, _RPW), jnp.int32),
            pltpu.VMEM((3, _RPW, _D), jnp.float32),
            pltpu.VMEM((_RPW, _D), jnp.float32),
            pltpu.SemaphoreType.DMA,
        ],
    )(_sc_body)


@jax.jit
def kernel(x):
    outs = []
    for p in range(_NPART):
        iw = _topk_keys(x, p)
        outs.append(_sc_interpolate()(iw, x))
    return outs[0] if _NPART == 1 else jnp.concatenate(outs, axis=0)


# speedup vs baseline: 1.2872x; 1.0155x over previous
import functools

import jax
import jax.numpy as jnp
from jax import lax
from jax.experimental import pallas as pl
from jax.experimental.pallas import tpu as pltpu
from jax.experimental.pallas import tpu_sc as plsc

_N, _D = 4096, 128
_BQ = 512
_IDXM = 4095
_NW = 32
_NPART = 1
_QP = _N // _NPART
_RPW = _QP // _NW


_BIAS = 1 << 23


def _keys_body(q_ref, k_ref, o_ref, sqk_ref, *, qoff=0):
    qi = pl.program_id(0) + qoff
    q = q_ref[...]
    k = k_ref[...]

    @pl.when(qi == 0)
    def _():
        sqk_ref[...] = jnp.sum(k * k, axis=1, keepdims=True)

    g = lax.dot_general(
        k, q * 2.0, (((1,), (1,)), ((), ())),
        preferred_element_type=jnp.float32)
    sq_q = jnp.sum(q * q, axis=1, keepdims=True).T
    d2 = (sqk_ref[...] - g) + sq_q

    rows = lax.broadcasted_iota(jnp.int32, (k.shape[0], 1), 0)
    cols = lax.broadcasted_iota(jnp.int32, (1, _BQ), 1) + qi * _BQ

    bits = lax.bitcast_convert_type(d2, jnp.int32)
    keys = lax.bitcast_convert_type(
        (bits & jnp.int32(~_IDXM)) + (rows + _BIAS), jnp.float32)
    inf = jnp.float32(jnp.inf)
    k2 = jnp.where(rows == cols, inf, keys)
    half = k.shape[0] // 2
    lo = k2[:half]
    hi = k2[half:]
    fmn = jnp.minimum(lo, hi)
    fmx = jnp.maximum(lo, hi)
    m2 = jnp.min(fmn, axis=0, keepdims=True)
    h = jnp.where(fmn == m2, fmx, fmn)
    m3 = jnp.min(h, axis=0, keepdims=True)

    def unpack(m):
        mb = lax.bitcast_convert_type(m, jnp.int32) - _BIAS
        d2m = lax.bitcast_convert_type(mb & jnp.int32(~_IDXM), jnp.float32)
        idx = (mb & jnp.int32(_IDXM)).astype(jnp.float32)
        return idx, 1.0 / jnp.maximum(d2m, 1e-16)

    i2, w2 = unpack(m2)
    i3, w3 = unpack(m3)
    w1 = jnp.full(i2.shape, 1e16, jnp.float32)
    inv = 1.0 / (w1 + w2 + w3)
    o_ref[0, 0:1, :] = cols.astype(jnp.float32)
    o_ref[0, 1:2, :] = i2
    o_ref[0, 2:3, :] = i3
    o_ref[0, 3:4, :] = w1 * inv
    o_ref[0, 4:5, :] = w2 * inv
    o_ref[0, 5:6, :] = w3 * inv


def _topk_keys(x, part):
    n, d = x.shape
    nblk = _QP // _BQ
    return pl.pallas_call(
        functools.partial(_keys_body, qoff=part * nblk),
        grid=(nblk,),
        in_specs=[
            pl.BlockSpec((_BQ, d), lambda i: (i + part * nblk, 0)),
            pl.BlockSpec((n, d), lambda i: (0, 0)),
        ],
        out_specs=pl.BlockSpec((1, 6, _BQ), lambda i: (i, 0, 0)),
        out_shape=jax.ShapeDtypeStruct((nblk, 6, _BQ), jnp.float32),
        scratch_shapes=[pltpu.VMEM((n, 1), jnp.float32)],
    )(x, x)


def _sc_body(iw_hbm, x_hbm, out_hbm, stage_v, idx_v, g_v, out_v, sem):
    wid = lax.axis_index("s") * 2 + lax.axis_index("c")
    base = wid * _RPW
    blk = base // _BQ
    off = base % _BQ

    pltpu.sync_copy(
        iw_hbm.at[pl.ds(blk, 1), pl.ds(0, 6), pl.ds(off, _RPW)], stage_v)

    for j in (1, 2):
        for s in range(_RPW // 16):
            sl = pl.ds(s * 16, 16)
            idx_v[j, sl] = stage_v[0, j, sl].astype(jnp.int32)

    copies = [pltpu.async_copy(x_hbm.at[pl.ds(base, _RPW)], g_v.at[0], sem)]
    copies += [
        pltpu.async_copy(x_hbm.at[idx_v.at[j]], g_v.at[j], sem)
        for j in (1, 2)
    ]
    for c in copies:
        c.wait()

    def group_body(g, carry):
        gb = g * 16
        wa = [stage_v[0, 3 + j, pl.ds(gb, 16)] for j in range(3)]
        for i in range(16):
            r = gb + i
            a0, a1, a2 = wa[0][i], wa[1][i], wa[2][i]
            for s in range(_D // 16):
                sl = pl.ds(s * 16, 16)
                out_v[r, sl] = (g_v[0, r, sl] * a0 + g_v[1, r, sl] * a1
                                + g_v[2, r, sl] * a2)
        return carry

    lax.fori_loop(0, _RPW // 16, group_body, 0)

    pltpu.sync_copy(out_v, out_hbm.at[pl.ds(base, _RPW)])


@functools.cache
def _sc_interpolate():
    return functools.partial(
        pl.kernel,
        mesh=plsc.VectorSubcoreMesh(core_axis_name="c", subcore_axis_name="s"),
        out_type=jax.ShapeDtypeStruct((_QP, _D), jnp.float32),
        scratch_types=[
            pltpu.VMEM((1, 6, _RPW), jnp.float32),
            pltpu.VMEM((3, _RPW), jnp.int32),
            pltpu.VMEM((3, _RPW, _D), jnp.float32),
            pltpu.VMEM((_RPW, _D), jnp.float32),
            pltpu.SemaphoreType.DMA,
        ],
    )(_sc_body)


@jax.jit
def kernel(x):
    outs = []
    for p in range(_NPART):
        iw = _topk_keys(x, p)
        outs.append(_sc_interpolate()(iw, x))
    return outs[0] if _NPART == 1 else jnp.concatenate(outs, axis=0)


# speedup vs baseline: 1.3008x; 1.0106x over previous
import functools

import jax
import jax.numpy as jnp
from jax import lax
from jax.experimental import pallas as pl
from jax.experimental.pallas import tpu as pltpu
from jax.experimental.pallas import tpu_sc as plsc

_N, _D = 4096, 128
_BQ = 512
_IDXM = 4095
_NW = 32
_NPART = 1
_QP = _N // _NPART
_RPW = _QP // _NW


_BIAS = 1 << 23


def _keys_body(q_ref, k_ref, o_ref, sqk_ref, *, qoff=0):
    qi = pl.program_id(0) + qoff
    q = q_ref[...]
    k = k_ref[...]

    @pl.when(qi == 0)
    def _():
        sqk_ref[...] = jnp.sum(k * k, axis=1, keepdims=True)

    g = lax.dot_general(
        k, q * 2.0, (((1,), (1,)), ((), ())),
        preferred_element_type=jnp.float32)
    sq_q = jnp.sum(q * q, axis=1, keepdims=True).T
    d2 = (sqk_ref[...] - g) + sq_q

    rows = lax.broadcasted_iota(jnp.int32, (k.shape[0], 1), 0)
    cols = lax.broadcasted_iota(jnp.int32, (1, _BQ), 1) + qi * _BQ

    bits = lax.bitcast_convert_type(d2, jnp.int32)
    keys = lax.bitcast_convert_type(
        (bits & jnp.int32(~_IDXM)) + (rows + _BIAS), jnp.float32)
    inf = jnp.float32(jnp.inf)
    k2 = jnp.where(rows == cols, inf, keys)
    half = k.shape[0] // 2
    lo = k2[:half]
    hi = k2[half:]
    fmn = jnp.minimum(lo, hi)
    fmx = jnp.maximum(lo, hi)
    m2 = jnp.min(fmn, axis=0, keepdims=True)
    h = jnp.where(fmn == m2, fmx, fmn)
    m3 = jnp.min(h, axis=0, keepdims=True)

    def unpack(m):
        mb = lax.bitcast_convert_type(m, jnp.int32) - _BIAS
        d2m = lax.bitcast_convert_type(mb & jnp.int32(~_IDXM), jnp.float32)
        idx = (mb & jnp.int32(_IDXM)).astype(jnp.float32)
        return idx, 1.0 / jnp.maximum(d2m, 1e-16)

    i2, w2 = unpack(m2)
    i3, w3 = unpack(m3)
    w1 = jnp.full(i2.shape, 1e16, jnp.float32)
    inv = 1.0 / (w1 + w2 + w3)
    o_ref[0, 0:1, :] = cols.astype(jnp.float32)
    o_ref[0, 1:2, :] = i2
    o_ref[0, 2:3, :] = i3
    o_ref[0, 3:4, :] = w1 * inv
    o_ref[0, 4:5, :] = w2 * inv
    o_ref[0, 5:6, :] = w3 * inv


def _topk_keys(x, part):
    n, d = x.shape
    nblk = _QP // _BQ
    return pl.pallas_call(
        functools.partial(_keys_body, qoff=part * nblk),
        grid=(nblk,),
        in_specs=[
            pl.BlockSpec((_BQ, d), lambda i: (i + part * nblk, 0)),
            pl.BlockSpec((n, d), lambda i: (0, 0)),
        ],
        out_specs=pl.BlockSpec((1, 6, _BQ), lambda i: (i, 0, 0)),
        out_shape=jax.ShapeDtypeStruct((nblk, 6, _BQ), jnp.float32),
        scratch_shapes=[pltpu.VMEM((n, 1), jnp.float32)],
    )(x, x)


def _sc_body(iw_hbm, x_hbm, out_hbm, stage_v, idx_v, g_v, out_v, sem):
    wid = lax.axis_index("s") * 2 + lax.axis_index("c")
    base = wid * _RPW
    blk = base // _BQ
    off = base % _BQ

    copies = [pltpu.async_copy(x_hbm.at[pl.ds(base, _RPW)], g_v.at[0], sem)]

    pltpu.sync_copy(
        iw_hbm.at[pl.ds(blk, 1), pl.ds(0, 6), pl.ds(off, _RPW)], stage_v)

    for j in (1, 2):
        for s in range(_RPW // 16):
            sl = pl.ds(s * 16, 16)
            idx_v[j, sl] = stage_v[0, j, sl].astype(jnp.int32)

    copies += [
        pltpu.async_copy(x_hbm.at[idx_v.at[j]], g_v.at[j], sem)
        for j in (1, 2)
    ]
    for c in copies:
        c.wait()

    def group_body(g, carry):
        gb = g * 16
        wa = [stage_v[0, 3 + j, pl.ds(gb, 16)] for j in range(3)]
        for i in range(16):
            r = gb + i
            a0, a1, a2 = wa[0][i], wa[1][i], wa[2][i]
            for s in range(_D // 16):
                sl = pl.ds(s * 16, 16)
                out_v[r, sl] = (g_v[0, r, sl] * a0 + g_v[1, r, sl] * a1
                                + g_v[2, r, sl] * a2)
        return carry

    lax.fori_loop(0, _RPW // 16, group_body, 0)

    pltpu.sync_copy(out_v, out_hbm.at[pl.ds(base, _RPW)])


@functools.cache
def _sc_interpolate():
    return functools.partial(
        pl.kernel,
        mesh=plsc.VectorSubcoreMesh(core_axis_name="c", subcore_axis_name="s"),
        out_type=jax.ShapeDtypeStruct((_QP, _D), jnp.float32),
        scratch_types=[
            pltpu.VMEM((1, 6, _RPW), jnp.float32),
            pltpu.VMEM((3, _RPW), jnp.int32),
            pltpu.VMEM((3, _RPW, _D), jnp.float32),
            pltpu.VMEM((_RPW, _D), jnp.float32),
            pltpu.SemaphoreType.DMA,
        ],
    )(_sc_body)


@jax.jit
def kernel(x):
    outs = []
    for p in range(_NPART):
        iw = _topk_keys(x, p)
        outs.append(_sc_interpolate()(iw, x))
    return outs[0] if _NPART == 1 else jnp.concatenate(outs, axis=0)


# speedup vs baseline: 1.3370x; 1.0278x over previous
import functools

import jax
import jax.numpy as jnp
from jax import lax
from jax.experimental import pallas as pl
from jax.experimental.pallas import tpu as pltpu
from jax.experimental.pallas import tpu_sc as plsc

_N, _D = 4096, 128
_BQ = 1024
_IDXM = 4095
_NW = 32
_NPART = 1
_QP = _N // _NPART
_RPW = _QP // _NW


_BIAS = 1 << 23


def _keys_body(q_ref, k_ref, o_ref, sqk_ref, *, qoff=0):
    qi = pl.program_id(0) + qoff
    q = q_ref[...]
    k = k_ref[...]

    @pl.when(qi == 0)
    def _():
        sqk_ref[...] = jnp.sum(k * k, axis=1, keepdims=True)

    g = lax.dot_general(
        k, q * 2.0, (((1,), (1,)), ((), ())),
        preferred_element_type=jnp.float32)
    sq_q = jnp.sum(q * q, axis=1, keepdims=True).T
    d2 = (sqk_ref[...] - g) + sq_q

    rows = lax.broadcasted_iota(jnp.int32, (k.shape[0], 1), 0)
    cols = lax.broadcasted_iota(jnp.int32, (1, _BQ), 1) + qi * _BQ

    bits = lax.bitcast_convert_type(d2, jnp.int32)
    keys = lax.bitcast_convert_type(
        (bits & jnp.int32(~_IDXM)) + (rows + _BIAS), jnp.float32)
    inf = jnp.float32(jnp.inf)
    k2 = jnp.where(rows == cols, inf, keys)
    half = k.shape[0] // 2
    lo = k2[:half]
    hi = k2[half:]
    fmn = jnp.minimum(lo, hi)
    fmx = jnp.maximum(lo, hi)
    m2 = jnp.min(fmn, axis=0, keepdims=True)
    h = jnp.where(fmn == m2, fmx, fmn)
    m3 = jnp.min(h, axis=0, keepdims=True)

    def unpack(m):
        mb = lax.bitcast_convert_type(m, jnp.int32) - _BIAS
        d2m = lax.bitcast_convert_type(mb & jnp.int32(~_IDXM), jnp.float32)
        idx = (mb & jnp.int32(_IDXM)).astype(jnp.float32)
        return idx, 1.0 / jnp.maximum(d2m, 1e-16)

    i2, w2 = unpack(m2)
    i3, w3 = unpack(m3)
    w1 = jnp.full(i2.shape, 1e16, jnp.float32)
    inv = 1.0 / (w1 + w2 + w3)
    o_ref[0, 0:1, :] = cols.astype(jnp.float32)
    o_ref[0, 1:2, :] = i2
    o_ref[0, 2:3, :] = i3
    o_ref[0, 3:4, :] = w1 * inv
    o_ref[0, 4:5, :] = w2 * inv
    o_ref[0, 5:6, :] = w3 * inv


def _topk_keys(x, part):
    n, d = x.shape
    nblk = _QP // _BQ
    return pl.pallas_call(
        functools.partial(_keys_body, qoff=part * nblk),
        grid=(nblk,),
        in_specs=[
            pl.BlockSpec((_BQ, d), lambda i: (i + part * nblk, 0)),
            pl.BlockSpec((n, d), lambda i: (0, 0)),
        ],
        out_specs=pl.BlockSpec((1, 6, _BQ), lambda i: (i, 0, 0)),
        out_shape=jax.ShapeDtypeStruct((nblk, 6, _BQ), jnp.float32),
        scratch_shapes=[pltpu.VMEM((n, 1), jnp.float32)],
    )(x, x)


def _sc_body(iw_hbm, x_hbm, out_hbm, stage_v, idx_v, g_v, out_v, sem):
    wid = lax.axis_index("s") * 2 + lax.axis_index("c")
    base = wid * _RPW
    blk = base // _BQ
    off = base % _BQ

    copies = [pltpu.async_copy(x_hbm.at[pl.ds(base, _RPW)], g_v.at[0], sem)]

    pltpu.sync_copy(
        iw_hbm.at[pl.ds(blk, 1), pl.ds(0, 6), pl.ds(off, _RPW)], stage_v)

    for j in (1, 2):
        for s in range(_RPW // 16):
            sl = pl.ds(s * 16, 16)
            idx_v[j, sl] = stage_v[0, j, sl].astype(jnp.int32)

    copies += [
        pltpu.async_copy(x_hbm.at[idx_v.at[j]], g_v.at[j], sem)
        for j in (1, 2)
    ]
    for c in copies:
        c.wait()

    def group_body(g, carry):
        gb = g * 16
        wa = [stage_v[0, 3 + j, pl.ds(gb, 16)] for j in range(3)]
        for i in range(16):
            r = gb + i
            a0, a1, a2 = wa[0][i], wa[1][i], wa[2][i]
            for s in range(_D // 16):
                sl = pl.ds(s * 16, 16)
                out_v[r, sl] = (g_v[0, r, sl] * a0 + g_v[1, r, sl] * a1
                                + g_v[2, r, sl] * a2)
        return carry

    lax.fori_loop(0, _RPW // 16, group_body, 0)

    pltpu.sync_copy(out_v, out_hbm.at[pl.ds(base, _RPW)])


@functools.cache
def _sc_interpolate():
    return functools.partial(
        pl.kernel,
        mesh=plsc.VectorSubcoreMesh(core_axis_name="c", subcore_axis_name="s"),
        out_type=jax.ShapeDtypeStruct((_QP, _D), jnp.float32),
        scratch_types=[
            pltpu.VMEM((1, 6, _RPW), jnp.float32),
            pltpu.VMEM((3, _RPW), jnp.int32),
            pltpu.VMEM((3, _RPW, _D), jnp.float32),
            pltpu.VMEM((_RPW, _D), jnp.float32),
            pltpu.SemaphoreType.DMA,
        ],
    )(_sc_body)


@jax.jit
def kernel(x):
    outs = []
    for p in range(_NPART):
        iw = _topk_keys(x, p)
        outs.append(_sc_interpolate()(iw, x))
    return outs[0] if _NPART == 1 else jnp.concatenate(outs, axis=0)


# speedup vs baseline: 1.3599x; 1.0172x over previous
import functools

import jax
import jax.numpy as jnp
from jax import lax
from jax.experimental import pallas as pl
from jax.experimental.pallas import tpu as pltpu
from jax.experimental.pallas import tpu_sc as plsc

_N, _D = 4096, 128
_BQ = 2048
_IDXM = 4095
_NW = 32
_NPART = 1
_QP = _N // _NPART
_RPW = _QP // _NW


_BIAS = 1 << 23


def _keys_body(q_ref, k_ref, o_ref, sqk_ref, *, qoff=0):
    qi = pl.program_id(0) + qoff
    q = q_ref[...]
    k = k_ref[...]

    @pl.when(qi == 0)
    def _():
        sqk_ref[...] = jnp.sum(k * k, axis=1, keepdims=True)

    g = lax.dot_general(
        k, q * 2.0, (((1,), (1,)), ((), ())),
        preferred_element_type=jnp.float32)
    sq_q = jnp.sum(q * q, axis=1, keepdims=True).T
    d2 = (sqk_ref[...] - g) + sq_q

    rows = lax.broadcasted_iota(jnp.int32, (k.shape[0], 1), 0)
    cols = lax.broadcasted_iota(jnp.int32, (1, _BQ), 1) + qi * _BQ

    bits = lax.bitcast_convert_type(d2, jnp.int32)
    keys = lax.bitcast_convert_type(
        (bits & jnp.int32(~_IDXM)) + (rows + _BIAS), jnp.float32)
    inf = jnp.float32(jnp.inf)
    k2 = jnp.where(rows == cols, inf, keys)
    half = k.shape[0] // 2
    lo = k2[:half]
    hi = k2[half:]
    fmn = jnp.minimum(lo, hi)
    fmx = jnp.maximum(lo, hi)
    m2 = jnp.min(fmn, axis=0, keepdims=True)
    h = jnp.where(fmn == m2, fmx, fmn)
    m3 = jnp.min(h, axis=0, keepdims=True)

    def unpack(m):
        mb = lax.bitcast_convert_type(m, jnp.int32) - _BIAS
        d2m = lax.bitcast_convert_type(mb & jnp.int32(~_IDXM), jnp.float32)
        idx = (mb & jnp.int32(_IDXM)).astype(jnp.float32)
        return idx, 1.0 / jnp.maximum(d2m, 1e-16)

    i2, w2 = unpack(m2)
    i3, w3 = unpack(m3)
    w1 = jnp.full(i2.shape, 1e16, jnp.float32)
    inv = 1.0 / (w1 + w2 + w3)
    o_ref[0, 0:1, :] = cols.astype(jnp.float32)
    o_ref[0, 1:2, :] = i2
    o_ref[0, 2:3, :] = i3
    o_ref[0, 3:4, :] = w1 * inv
    o_ref[0, 4:5, :] = w2 * inv
    o_ref[0, 5:6, :] = w3 * inv


def _topk_keys(x, part):
    n, d = x.shape
    nblk = _QP // _BQ
    return pl.pallas_call(
        functools.partial(_keys_body, qoff=part * nblk),
        grid=(nblk,),
        in_specs=[
            pl.BlockSpec((_BQ, d), lambda i: (i + part * nblk, 0)),
            pl.BlockSpec((n, d), lambda i: (0, 0)),
        ],
        out_specs=pl.BlockSpec((1, 6, _BQ), lambda i: (i, 0, 0)),
        out_shape=jax.ShapeDtypeStruct((nblk, 6, _BQ), jnp.float32),
        scratch_shapes=[pltpu.VMEM((n, 1), jnp.float32)],
    )(x, x)


def _sc_body(iw_hbm, x_hbm, out_hbm, stage_v, idx_v, g_v, out_v, sem):
    wid = lax.axis_index("s") * 2 + lax.axis_index("c")
    base = wid * _RPW
    blk = base // _BQ
    off = base % _BQ

    copies = [pltpu.async_copy(x_hbm.at[pl.ds(base, _RPW)], g_v.at[0], sem)]

    pltpu.sync_copy(
        iw_hbm.at[pl.ds(blk, 1), pl.ds(0, 6), pl.ds(off, _RPW)], stage_v)

    for j in (1, 2):
        for s in range(_RPW // 16):
            sl = pl.ds(s * 16, 16)
            idx_v[j, sl] = stage_v[0, j, sl].astype(jnp.int32)

    copies += [
        pltpu.async_copy(x_hbm.at[idx_v.at[j]], g_v.at[j], sem)
        for j in (1, 2)
    ]
    for c in copies:
        c.wait()

    def group_body(g, carry):
        gb = g * 16
        wa = [stage_v[0, 3 + j, pl.ds(gb, 16)] for j in range(3)]
        for i in range(16):
            r = gb + i
            a0, a1, a2 = wa[0][i], wa[1][i], wa[2][i]
            for s in range(_D // 16):
                sl = pl.ds(s * 16, 16)
                out_v[r, sl] = (g_v[0, r, sl] * a0 + g_v[1, r, sl] * a1
                                + g_v[2, r, sl] * a2)
        return carry

    lax.fori_loop(0, _RPW // 16, group_body, 0)

    pltpu.sync_copy(out_v, out_hbm.at[pl.ds(base, _RPW)])


@functools.cache
def _sc_interpolate():
    return functools.partial(
        pl.kernel,
        mesh=plsc.VectorSubcoreMesh(core_axis_name="c", subcore_axis_name="s"),
        out_type=jax.ShapeDtypeStruct((_QP, _D), jnp.float32),
        scratch_types=[
            pltpu.VMEM((1, 6, _RPW), jnp.float32),
            pltpu.VMEM((3, _RPW), jnp.int32),
            pltpu.VMEM((3, _RPW, _D), jnp.float32),
            pltpu.VMEM((_RPW, _D), jnp.float32),
            pltpu.SemaphoreType.DMA,
        ],
    )(_sc_body)


@jax.jit
def kernel(x):
    outs = []
    for p in range(_NPART):
        iw = _topk_keys(x, p)
        outs.append(_sc_interpolate()(iw, x))
    return outs[0] if _NPART == 1 else jnp.concatenate(outs, axis=0)
